# baseline (speedup 1.0000x reference)
import functools

import jax
import jax.numpy as jnp
from jax.experimental import pallas as pl
from jax.experimental.pallas import tpu as pltpu

B, N, D, H = 64, 1000000, 64, 512
TN = 20000
NUM_TILES = N // TN
SUB = 32
SPT = TN // SUB
SUBN = N // SUB


def _scan_body(codes_ref, table_ref, subidx_ref, minval_ref,
               stat_ref, best_ref, bidx_ref):
    t = pl.program_id(0)

    @pl.when(t == 0)
    def _():
        stat_ref[0:D, :] = jnp.ones((D, B), jnp.float32)
        stat_ref[D:2 * D, :] = codes_ref[...] * (-2.0)

    tile = table_ref[...]
    aug = jnp.concatenate([tile * tile, tile], axis=1)
    scores = jax.lax.dot_general(
        aug, stat_ref[...], (((1,), (0,)), ((), ())),
        preferred_element_type=jnp.float32)
    tmins = jnp.min(scores.reshape(SPT, SUB, B), axis=1)
    tm = jnp.min(tmins, axis=0, keepdims=True)
    si = jax.lax.broadcasted_iota(
        jnp.int32, (SPT, B), 0).astype(jnp.float32)
    targ = jnp.min(jnp.where(tmins == tm, si, jnp.float32(1e9)),
                   axis=0, keepdims=True)
    gtarg = targ + jnp.float32(SPT) * t.astype(jnp.float32)

    @pl.when(t == 0)
    def _():
        best_ref[...] = tm
        bidx_ref[...] = gtarg

    @pl.when(t > 0)
    def _():
        upd = tm < best_ref[...]
        best_ref[...] = jnp.where(upd, tm, best_ref[...])
        bidx_ref[...] = jnp.where(upd, gtarg, bidx_ref[...])

    @pl.when(t == NUM_TILES - 1)
    def _():
        subidx_ref[...] = bidx_ref[...].astype(jnp.int32)
        minval_ref[...] = best_ref[...]


def _rescore_body(subidx_smem, subidx_vec_ref, minval_ref, codes_ref,
                  table_ref, W1_ref, b1_ref, W2u_ref, b2u_ref,
                  W2s_ref, b2s_ref, mu_ref, ls_ref,
                  gat_ref, stat_ref, sem):
    copies = []
    for b in range(B):
        c = pltpu.make_async_copy(
            table_ref.at[pl.ds(subidx_smem[0, b] * SUB, SUB), :],
            gat_ref.at[pl.ds(b * SUB, SUB), :], sem)
        c.start()
        copies.append(c)
    stat_ref[0:D, :] = jnp.ones((D, B), jnp.float32)
    stat_ref[D:2 * D, :] = codes_ref[...] * (-2.0)
    for c in copies:
        c.wait()
    g = gat_ref[...]
    aug = jnp.concatenate([g * g, g], axis=1)
    scores = jax.lax.dot_general(
        aug, stat_ref[...], (((1,), (0,)), ((), ())),
        preferred_element_type=jnp.float32)
    subvT = subidx_vec_ref[...].T
    base3 = jnp.broadcast_to(subvT.reshape(B, 1, 1), (B, SUB, B))
    r3 = jax.lax.broadcasted_iota(jnp.int32, (B, SUB, B), 1)
    tbl = (base3 * SUB + r3).reshape(B * SUB, B).astype(jnp.float32)
    eq = scores == minval_ref[...]
    rmin = jnp.min(jnp.where(eq, tbl, jnp.float32(2e9)),
                   axis=0, keepdims=True)
    bm3 = (jax.lax.broadcasted_iota(jnp.int32, (B, SUB, B), 0)
           == jax.lax.broadcasted_iota(jnp.int32, (B, SUB, B), 2))
    oh = jnp.where(jnp.logical_and(bm3.reshape(B * SUB, B), tbl == rmin),
                   jnp.float32(1.0), jnp.float32(0.0))
    sel = jax.lax.dot_general(
        oh, g, (((0,), (0,)), ((), ())),
        preferred_element_type=jnp.float32)
    h1 = jax.lax.dot_general(
        sel, W1_ref[...], (((1,), (1,)), ((), ())),
        preferred_element_type=jnp.float32) + b1_ref[...]
    h1 = jnp.maximum(h1, 0.0)
    mu_ref[...] = jax.lax.dot_general(
        h1, W2u_ref[...], (((1,), (1,)), ((), ())),
        preferred_element_type=jnp.float32) + b2u_ref[...]
    ls_ref[...] = jax.lax.dot_general(
        h1, W2s_ref[...], (((1,), (1,)), ((), ())),
        preferred_element_type=jnp.float32) + b2s_ref[...]


@functools.partial(jax.jit, static_argnames=("interpret",))
def kernel(codes, codes_table, W1, b1, W2u, b2u, W2s, b2s, interpret=False):
    codesT = codes.T
    subidx, minval = pl.pallas_call(
        _scan_body,
        grid=(NUM_TILES,),
        in_specs=[
            pl.BlockSpec((D, B), lambda t: (0, 0)),
            pl.BlockSpec((TN, D), lambda t: (t, 0)),
        ],
        out_specs=(pl.BlockSpec((1, B), lambda t: (0, 0)),
                   pl.BlockSpec((1, B), lambda t: (0, 0))),
        out_shape=(jax.ShapeDtypeStruct((1, B), jnp.int32),
                   jax.ShapeDtypeStruct((1, B), jnp.float32)),
        scratch_shapes=[
            pltpu.VMEM((2 * D, B), jnp.float32),
            pltpu.VMEM((1, B), jnp.float32),
            pltpu.VMEM((1, B), jnp.float32),
        ],
        interpret=interpret,
    )(codesT, codes_table)

    mu, logstd = pl.pallas_call(
        _rescore_body,
        in_specs=[
            pl.BlockSpec(memory_space=pltpu.SMEM),
            pl.BlockSpec(memory_space=pltpu.VMEM),
            pl.BlockSpec(memory_space=pltpu.VMEM),
            pl.BlockSpec(memory_space=pltpu.VMEM),
            pl.BlockSpec(memory_space=pl.ANY),
            pl.BlockSpec(memory_space=pltpu.VMEM),
            pl.BlockSpec(memory_space=pltpu.VMEM),
            pl.BlockSpec(memory_space=pltpu.VMEM),
            pl.BlockSpec(memory_space=pltpu.VMEM),
            pl.BlockSpec(memory_space=pltpu.VMEM),
            pl.BlockSpec(memory_space=pltpu.VMEM),
        ],
        out_specs=(pl.BlockSpec(memory_space=pltpu.VMEM),
                   pl.BlockSpec(memory_space=pltpu.VMEM)),
        out_shape=(jax.ShapeDtypeStruct((B, D), jnp.float32),
                   jax.ShapeDtypeStruct((B, D), jnp.float32)),
        scratch_shapes=[
            pltpu.VMEM((B * SUB, D), jnp.float32),
            pltpu.VMEM((2 * D, B), jnp.float32),
            pltpu.SemaphoreType.DMA,
        ],
        interpret=interpret,
    )(subidx, subidx, minval, codesT, codes_table,
      W1, b1.reshape(1, H), W2u, b2u.reshape(1, D), W2s, b2s.reshape(1, D))
    return (mu, logstd)


# speedup vs baseline: 1.0261x; 1.0261x over previous
import functools

import jax
import jax.numpy as jnp
from jax.experimental import pallas as pl
from jax.experimental.pallas import tpu as pltpu

B, N, D, H = 64, 1000000, 64, 512
TN = 20000
NUM_TILES = N // TN
SUB = 160
SPT = TN // SUB
SUBN = N // SUB


def _scan_body(codes_ref, table_ref, subidx_ref, minval_ref,
               stat_ref, best_ref, bidx_ref):
    t = pl.program_id(0)

    @pl.when(t == 0)
    def _():
        stat_ref[0:D, :] = jnp.ones((D, B), jnp.float32)
        stat_ref[D:2 * D, :] = codes_ref[...] * (-2.0)

    tile = table_ref[...]
    aug = jnp.concatenate([tile * tile, tile], axis=1)
    scores = jax.lax.dot_general(
        aug, stat_ref[...], (((1,), (0,)), ((), ())),
        preferred_element_type=jnp.float32)
    tmins = jnp.min(scores.reshape(SPT, SUB, B), axis=1)
    tm = jnp.min(tmins, axis=0, keepdims=True)
    si = jax.lax.broadcasted_iota(
        jnp.int32, (SPT, B), 0).astype(jnp.float32)
    targ = jnp.min(jnp.where(tmins == tm, si, jnp.float32(1e9)),
                   axis=0, keepdims=True)
    gtarg = targ + jnp.float32(SPT) * t.astype(jnp.float32)

    @pl.when(t == 0)
    def _():
        best_ref[...] = tm
        bidx_ref[...] = gtarg

    @pl.when(t > 0)
    def _():
        upd = tm < best_ref[...]
        best_ref[...] = jnp.where(upd, tm, best_ref[...])
        bidx_ref[...] = jnp.where(upd, gtarg, bidx_ref[...])

    @pl.when(t == NUM_TILES - 1)
    def _():
        subidx_ref[...] = bidx_ref[...].astype(jnp.int32)
        minval_ref[...] = best_ref[...]


def _rescore_body(subidx_smem, subidx_vec_ref, minval_ref, codes_ref,
                  table_ref, W1_ref, b1_ref, W2u_ref, b2u_ref,
                  W2s_ref, b2s_ref, mu_ref, ls_ref,
                  gat_ref, stat_ref, sem):
    copies = []
    for b in range(B):
        c = pltpu.make_async_copy(
            table_ref.at[pl.ds(subidx_smem[0, b] * SUB, SUB), :],
            gat_ref.at[pl.ds(b * SUB, SUB), :], sem)
        c.start()
        copies.append(c)
    stat_ref[0:D, :] = jnp.ones((D, B), jnp.float32)
    stat_ref[D:2 * D, :] = codes_ref[...] * (-2.0)
    for c in copies:
        c.wait()
    g = gat_ref[...]
    aug = jnp.concatenate([g * g, g], axis=1)
    scores = jax.lax.dot_general(
        aug, stat_ref[...], (((1,), (0,)), ((), ())),
        preferred_element_type=jnp.float32)
    subvT = subidx_vec_ref[...].T
    base3 = jnp.broadcast_to(subvT.reshape(B, 1, 1), (B, SUB, B))
    r3 = jax.lax.broadcasted_iota(jnp.int32, (B, SUB, B), 1)
    tbl = (base3 * SUB + r3).reshape(B * SUB, B).astype(jnp.float32)
    eq = scores == minval_ref[...]
    rmin = jnp.min(jnp.where(eq, tbl, jnp.float32(2e9)),
                   axis=0, keepdims=True)
    bm3 = (jax.lax.broadcasted_iota(jnp.int32, (B, SUB, B), 0)
           == jax.lax.broadcasted_iota(jnp.int32, (B, SUB, B), 2))
    oh = jnp.where(jnp.logical_and(bm3.reshape(B * SUB, B), tbl == rmin),
                   jnp.float32(1.0), jnp.float32(0.0))
    sel = jax.lax.dot_general(
        oh, g, (((0,), (0,)), ((), ())),
        preferred_element_type=jnp.float32)
    h1 = jax.lax.dot_general(
        sel, W1_ref[...], (((1,), (1,)), ((), ())),
        preferred_element_type=jnp.float32) + b1_ref[...]
    h1 = jnp.maximum(h1, 0.0)
    mu_ref[...] = jax.lax.dot_general(
        h1, W2u_ref[...], (((1,), (1,)), ((), ())),
        preferred_element_type=jnp.float32) + b2u_ref[...]
    ls_ref[...] = jax.lax.dot_general(
        h1, W2s_ref[...], (((1,), (1,)), ((), ())),
        preferred_element_type=jnp.float32) + b2s_ref[...]


@functools.partial(jax.jit, static_argnames=("interpret",))
def kernel(codes, codes_table, W1, b1, W2u, b2u, W2s, b2s, interpret=False):
    codesT = codes.T
    subidx, minval = pl.pallas_call(
        _scan_body,
        grid=(NUM_TILES,),
        in_specs=[
            pl.BlockSpec((D, B), lambda t: (0, 0)),
            pl.BlockSpec((TN, D), lambda t: (t, 0)),
        ],
        out_specs=(pl.BlockSpec((1, B), lambda t: (0, 0)),
                   pl.BlockSpec((1, B), lambda t: (0, 0))),
        out_shape=(jax.ShapeDtypeStruct((1, B), jnp.int32),
                   jax.ShapeDtypeStruct((1, B), jnp.float32)),
        scratch_shapes=[
            pltpu.VMEM((2 * D, B), jnp.float32),
            pltpu.VMEM((1, B), jnp.float32),
            pltpu.VMEM((1, B), jnp.float32),
        ],
        interpret=interpret,
    )(codesT, codes_table)

    mu, logstd = pl.pallas_call(
        _rescore_body,
        in_specs=[
            pl.BlockSpec(memory_space=pltpu.SMEM),
            pl.BlockSpec(memory_space=pltpu.VMEM),
            pl.BlockSpec(memory_space=pltpu.VMEM),
            pl.BlockSpec(memory_space=pltpu.VMEM),
            pl.BlockSpec(memory_space=pl.ANY),
            pl.BlockSpec(memory_space=pltpu.VMEM),
            pl.BlockSpec(memory_space=pltpu.VMEM),
            pl.BlockSpec(memory_space=pltpu.VMEM),
            pl.BlockSpec(memory_space=pltpu.VMEM),
            pl.BlockSpec(memory_space=pltpu.VMEM),
            pl.BlockSpec(memory_space=pltpu.VMEM),
        ],
        out_specs=(pl.BlockSpec(memory_space=pltpu.VMEM),
                   pl.BlockSpec(memory_space=pltpu.VMEM)),
        out_shape=(jax.ShapeDtypeStruct((B, D), jnp.float32),
                   jax.ShapeDtypeStruct((B, D), jnp.float32)),
        scratch_shapes=[
            pltpu.VMEM((B * SUB, D), jnp.float32),
            pltpu.VMEM((2 * D, B), jnp.float32),
            pltpu.SemaphoreType.DMA,
        ],
        interpret=interpret,
    )(subidx, subidx, minval, codesT, codes_table,
      W1, b1.reshape(1, H), W2u, b2u.reshape(1, D), W2s, b2s.reshape(1, D))
    return (mu, logstd)
